# baseline (speedup 1.0000x reference)
import functools

import jax
import jax.numpy as jnp
from jax import lax
from jax.experimental import pallas as pl
from jax.experimental.pallas import tpu as pltpu
from jax.experimental.pallas import tpu_sc as plsc

N = 10000
NP = 10240
G = 64
D = 128
H = 300
HP = 320
OUT = 256
BLK_D = 128
NSUB = 16
NCORE = 2
LANES = 16
RB = 512
F32 = jnp.float32


@functools.lru_cache(maxsize=None)
def _spmm_call(Wh, NB, BLK, CB):
    NBW = NB // NSUB
    NCH = NBW // CB
    RPW = NP // NSUB
    mesh = plsc.VectorSubcoreMesh(core_axis_name="c", subcore_axis_name="s")

    def body(t_hbm, src_hbm, dst_hbm, out_hbm, idx_s, idx_d, rb0, rb1,
             acc_sh, sem0, sem1):
        c = lax.axis_index("c")
        s = lax.axis_index("s")

        z = jnp.zeros((LANES,), F32)

        @pl.loop(0, BLK)
        def _(i):
            @pl.loop(0, Wh, step=LANES)
            def _(k):
                rb0[i, pl.ds(k, LANES)] = z

        @pl.loop(0, RPW, step=BLK)
        def _(r):
            pltpu.sync_copy(rb0, acc_sh.at[pl.ds(s * RPW + r, BLK)])

        plsc.subcore_barrier()

        def start(j, buf, sem):
            pltpu.async_copy(t_hbm.at[idx_s.at[j]], buf, sem)

        def wait(buf, sem):
            pltpu.make_async_copy(t_hbm.at[idx_s.at[0]], buf, sem).wait()

        def scat(j, buf):
            pltpu.sync_copy(buf, acc_sh.at[idx_d.at[j]], add=True)

        @pl.loop(0, NCH)
        def _(ch):
            base = s * NBW + ch * CB
            pltpu.sync_copy(src_hbm.at[c, pl.ds(base, CB)], idx_s)
            pltpu.sync_copy(dst_hbm.at[pl.ds(base, CB)], idx_d)

            start(0, rb0, sem0)

            @pl.loop(0, CB - 2, step=2)
            def _(j):
                start(j + 1, rb1, sem1)
                wait(rb0, sem0)
                scat(j, rb0)
                start(j + 2, rb0, sem0)
                wait(rb1, sem1)
                scat(j + 1, rb1)

            start(CB - 1, rb1, sem1)
            wait(rb0, sem0)
            scat(CB - 2, rb0)
            wait(rb1, sem1)
            scat(CB - 1, rb1)

        plsc.subcore_barrier()
        pltpu.sync_copy(acc_sh.at[pl.ds(s * RPW, RPW)],
                        out_hbm.at[c, pl.ds(s * RPW, RPW)])

    return pl.kernel(
        body,
        out_type=jax.ShapeDtypeStruct((NCORE, NP, Wh), F32),
        mesh=mesh,
        compiler_params=pltpu.CompilerParams(use_tc_tiling_on_sc=False),
        scratch_types=[
            pltpu.VMEM((CB, BLK), jnp.int32),
            pltpu.VMEM((CB, BLK), jnp.int32),
            pltpu.VMEM((BLK, Wh), F32),
            pltpu.VMEM((BLK, Wh), F32),
            pltpu.VMEM_SHARED((NP, Wh), F32),
            pltpu.SemaphoreType.DMA,
            pltpu.SemaphoreType.DMA,
        ],
    )


@functools.lru_cache(maxsize=None)
def _deg_call(NB):
    NBC = NB // (NCORE * NSUB)
    RPW = NP // NSUB
    mesh = plsc.VectorSubcoreMesh(core_axis_name="c", subcore_axis_name="s")

    def body(dst_hbm, out_hbm, idx_d, ones_b, deg_sh):
        c = lax.axis_index("c")
        s = lax.axis_index("s")
        z = jnp.zeros((LANES,), F32)

        @pl.loop(0, BLK_D)
        def _(i):
            ones_b[i, pl.ds(0, LANES)] = z

        @pl.loop(0, RPW, step=BLK_D)
        def _(r):
            pltpu.sync_copy(ones_b, deg_sh.at[pl.ds(s * RPW + r, BLK_D)])

        o = jnp.ones((LANES,), F32)

        @pl.loop(0, BLK_D)
        def _(i):
            ones_b[i, pl.ds(0, LANES)] = o

        plsc.subcore_barrier()
        pltpu.sync_copy(dst_hbm.at[pl.ds((c * NSUB + s) * NBC, NBC)], idx_d)

        @pl.loop(0, NBC)
        def _(j):
            pltpu.sync_copy(ones_b, deg_sh.at[idx_d.at[j]], add=True)

        plsc.subcore_barrier()
        pltpu.sync_copy(deg_sh.at[pl.ds(s * RPW, RPW)],
                        out_hbm.at[c, pl.ds(s * RPW, RPW)])

    return pl.kernel(
        body,
        out_type=jax.ShapeDtypeStruct((NCORE, NP, LANES), F32),
        mesh=mesh,
        compiler_params=pltpu.CompilerParams(use_tc_tiling_on_sc=False),
        scratch_types=[
            pltpu.VMEM((NBC, BLK_D), jnp.int32),
            pltpu.VMEM((BLK_D, LANES), F32),
            pltpu.VMEM_SHARED((NP, LANES), F32),
        ],
    )


def _dinv_of(deg_block):
    return lax.rsqrt(deg_block[0, :, 0] + deg_block[1, :, 0] + 1.0)


def _matmul1(xp, W1p):
    def body(x_ref, w_ref, o_ref):
        o_ref[...] = jnp.dot(x_ref[...], w_ref[...], preferred_element_type=F32)

    return pl.pallas_call(
        body,
        grid=(NP // RB,),
        in_specs=[pl.BlockSpec((RB, D), lambda i: (i, 0)),
                  pl.BlockSpec((D, HP), lambda i: (0, 0))],
        out_specs=pl.BlockSpec((RB, HP), lambda i: (i, 0)),
        out_shape=jax.ShapeDtypeStruct((NP, HP), F32),
    )(xp, W1p)


def _scale_split(t1, degp, Wh):
    def body(t_ref, deg_ref, o_ref):
        dinv = _dinv_of(deg_ref[...])
        tv = t_ref[...] * dinv[:, None]
        o_ref[0] = tv[:, :Wh]
        o_ref[1] = tv[:, Wh:]

    return pl.pallas_call(
        body,
        grid=(NP // RB,),
        in_specs=[pl.BlockSpec((RB, 2 * Wh), lambda i: (i, 0)),
                  pl.BlockSpec((NCORE, RB, LANES), lambda i: (0, i, 0))],
        out_specs=pl.BlockSpec((NCORE, RB, Wh), lambda i: (0, i, 0)),
        out_shape=jax.ShapeDtypeStruct((NCORE, NP, Wh), F32),
    )(t1, degp)


def _fuse_layer(acc, tp, degp, b_row, Wn, Wh_out):
    Wh_in = tp.shape[2]

    def body(a_ref, t_ref, deg_ref, b_ref, w_ref, o_ref):
        dinv = _dinv_of(deg_ref[...])
        a = jnp.concatenate([a_ref[0], a_ref[1]], axis=1)
        t = jnp.concatenate([t_ref[0], t_ref[1]], axis=1)
        h = jnp.maximum((a + t) * dinv[:, None] + b_ref[...], 0.0)
        tn = jnp.dot(h, w_ref[...], preferred_element_type=F32) * dinv[:, None]
        o_ref[0] = tn[:, :Wh_out]
        o_ref[1] = tn[:, Wh_out:]

    return pl.pallas_call(
        body,
        grid=(NP // RB,),
        in_specs=[pl.BlockSpec((NCORE, RB, Wh_in), lambda i: (0, i, 0)),
                  pl.BlockSpec((NCORE, RB, Wh_in), lambda i: (0, i, 0)),
                  pl.BlockSpec((NCORE, RB, LANES), lambda i: (0, i, 0)),
                  pl.BlockSpec((1, 2 * Wh_in), lambda i: (0, 0)),
                  pl.BlockSpec((2 * Wh_in, 2 * Wh_out), lambda i: (0, 0))],
        out_specs=pl.BlockSpec((NCORE, RB, Wh_out), lambda i: (0, i, 0)),
        out_shape=jax.ShapeDtypeStruct((NCORE, NP, Wh_out), F32),
    )(acc, tp, degp, b_row, Wn)


def _final(acc, tp, degp, b_row, batch3, Wm, bm):
    nb = NP // RB
    Wh_in = tp.shape[2]
    OUTP = 2 * Wh_in

    def body(a_ref, t_ref, deg_ref, b_ref, bt_ref, wm_ref, bm_ref, o_ref,
             sums, counts):
        i = pl.program_id(0)

        @pl.when(i == 0)
        def _():
            sums[...] = jnp.zeros_like(sums)
            counts[...] = jnp.zeros_like(counts)

        dinv = _dinv_of(deg_ref[...])
        a = jnp.concatenate([a_ref[0], a_ref[1]], axis=1)
        t = jnp.concatenate([t_ref[0], t_ref[1]], axis=1)
        h = jnp.maximum((a + t) * dinv[:, None] + b_ref[...], 0.0)
        bt = bt_ref[0, 0, :]
        gid = lax.broadcasted_iota(jnp.int32, (G, RB), 0)
        ind = (bt[None, :] == gid).astype(F32)
        sums[...] += jnp.dot(ind, h, preferred_element_type=F32)
        counts[...] += jnp.sum(ind, axis=1)[None, :]

        @pl.when(i == nb - 1)
        def _():
            ro = sums[...] / jnp.maximum(counts[0, :], 1.0)[:, None]
            o_ref[...] = jnp.dot(ro, wm_ref[...],
                                 preferred_element_type=F32) + bm_ref[0, 0]

    return pl.pallas_call(
        body,
        grid=(nb,),
        in_specs=[pl.BlockSpec((NCORE, RB, Wh_in), lambda i: (0, i, 0)),
                  pl.BlockSpec((NCORE, RB, Wh_in), lambda i: (0, i, 0)),
                  pl.BlockSpec((NCORE, RB, LANES), lambda i: (0, i, 0)),
                  pl.BlockSpec((1, OUTP), lambda i: (0, 0)),
                  pl.BlockSpec((1, 1, RB), lambda i: (i, 0, 0)),
                  pl.BlockSpec((OUTP, 1), lambda i: (0, 0)),
                  pl.BlockSpec((1, 1), lambda i: (0, 0))],
        out_specs=pl.BlockSpec((G, 1), lambda i: (0, 0)),
        out_shape=jax.ShapeDtypeStruct((G, 1), F32),
        scratch_shapes=[pltpu.VMEM((G, OUTP), F32), pltpu.VMEM((1, G), F32)],
    )(acc, tp, degp, b_row, batch3, Wm, bm)


def kernel(x, edge_index, batch, W1, b1, W2, b2, W3, b3, Wm, bm):
    E = edge_index.shape[1]
    CH = NCORE * NSUB * 128 * 8
    EP = -(-E // CH) * CH
    NB64 = EP // 64
    NB128 = EP // 128

    src = edge_index[0].astype(jnp.int32)
    dst = edge_index[1].astype(jnp.int32)
    pad = jnp.full((EP - E,), N, jnp.int32)
    srcf = jnp.concatenate([src, pad])
    dstf = jnp.concatenate([dst, pad])
    srcp64 = srcf.reshape(NB64, 64)
    dstp64 = dstf.reshape(NB64, 64)
    srcp128 = srcf.reshape(NB128, 128)
    dstp128 = dstf.reshape(NB128, 128)
    src2_64 = jnp.stack([srcp64, srcp64 + NP])
    src2_128 = jnp.stack([srcp128, srcp128 + NP])

    xp = jnp.pad(x.astype(F32), ((0, NP - N), (0, 0)))
    W1p = jnp.pad(W1, ((0, 0), (0, HP - H)))
    W2p = jnp.pad(W2, ((0, HP - H), (0, HP - H)))
    W3p = jnp.pad(W3, ((0, HP - H), (0, 0)))
    b1r = jnp.pad(b1, (0, HP - H))[None, :]
    b2r = jnp.pad(b2, (0, HP - H))[None, :]
    b3r = b3[None, :]
    batchp = jnp.concatenate(
        [batch.astype(jnp.int32), jnp.full((NP - N,), G, jnp.int32)])
    batch3 = batchp.reshape(NP // RB, 1, RB)

    degp = _deg_call(NB128)(dstp128)
    t1 = _matmul1(xp, W1p)
    t1p = _scale_split(t1, degp, HP // 2)
    acc1 = _spmm_call(HP // 2, NB64, 64, 32)(
        t1p.reshape(NCORE * NP, HP // 2), src2_64, dstp64)
    t2p = _fuse_layer(acc1, t1p, degp, b1r, W2p, HP // 2)
    acc2 = _spmm_call(HP // 2, NB64, 64, 32)(
        t2p.reshape(NCORE * NP, HP // 2), src2_64, dstp64)
    t3p = _fuse_layer(acc2, t2p, degp, b2r, W3p, OUT // 2)
    acc3 = _spmm_call(OUT // 2, NB64, 64, 32)(
        t3p.reshape(NCORE * NP, OUT // 2), src2_64, dstp64)
    out = _final(acc3, t3p, degp, b3r, batch3, Wm, bm[None, :])
    return out[:, 0]


# speedup vs baseline: 1.0400x; 1.0400x over previous
import functools

import jax
import jax.numpy as jnp
from jax import lax
from jax.experimental import pallas as pl
from jax.experimental.pallas import tpu as pltpu
from jax.experimental.pallas import tpu_sc as plsc

N = 10000
NP = 10240
G = 64
D = 128
H = 300
HP = 320
OUT = 256
BLK_D = 128
NSUB = 16
NCORE = 2
LANES = 16
RB = 512
F32 = jnp.float32
BF16 = jnp.bfloat16


@functools.lru_cache(maxsize=None)
def _spmm_call(Wh, NB, BLK, CB):
    NBW = NB // NSUB
    NCH = NBW // CB
    RPW = NP // NSUB
    mesh = plsc.VectorSubcoreMesh(core_axis_name="c", subcore_axis_name="s")

    def body(t_hbm, src_hbm, dst_hbm, out_hbm, idx_s, idx_d, rb0, rb1,
             acc_sh, sem0, sem1):
        c = lax.axis_index("c")
        s = lax.axis_index("s")

        z = jnp.zeros((LANES,), F32)

        @pl.loop(0, BLK)
        def _(i):
            @pl.loop(0, Wh, step=LANES)
            def _(k):
                rb0[i, pl.ds(k, LANES)] = z

        @pl.loop(0, RPW, step=BLK)
        def _(r):
            pltpu.sync_copy(rb0, acc_sh.at[pl.ds(s * RPW + r, BLK)])

        plsc.subcore_barrier()

        def start(j, buf, sem):
            pltpu.async_copy(t_hbm.at[idx_s.at[j]], buf, sem)

        def wait(buf, sem):
            pltpu.make_async_copy(t_hbm.at[idx_s.at[0]], buf, sem).wait()

        def scat(j, buf):
            pltpu.sync_copy(buf, acc_sh.at[idx_d.at[j]], add=True)

        @pl.loop(0, NCH)
        def _(ch):
            base = s * NBW + ch * CB
            pltpu.sync_copy(src_hbm.at[c, pl.ds(base, CB)], idx_s)
            pltpu.sync_copy(dst_hbm.at[pl.ds(base, CB)], idx_d)

            start(0, rb0, sem0)

            @pl.loop(0, CB - 2, step=2)
            def _(j):
                start(j + 1, rb1, sem1)
                wait(rb0, sem0)
                scat(j, rb0)
                start(j + 2, rb0, sem0)
                wait(rb1, sem1)
                scat(j + 1, rb1)

            start(CB - 1, rb1, sem1)
            wait(rb0, sem0)
            scat(CB - 2, rb0)
            wait(rb1, sem1)
            scat(CB - 1, rb1)

        plsc.subcore_barrier()
        pltpu.sync_copy(acc_sh.at[pl.ds(s * RPW, RPW)],
                        out_hbm.at[c, pl.ds(s * RPW, RPW)])

    return pl.kernel(
        body,
        out_type=jax.ShapeDtypeStruct((NCORE, NP, Wh), F32),
        mesh=mesh,
        compiler_params=pltpu.CompilerParams(use_tc_tiling_on_sc=False),
        scratch_types=[
            pltpu.VMEM((CB, BLK), jnp.int32),
            pltpu.VMEM((CB, BLK), jnp.int32),
            pltpu.VMEM((BLK, Wh), F32),
            pltpu.VMEM((BLK, Wh), F32),
            pltpu.VMEM_SHARED((NP, Wh), F32),
            pltpu.SemaphoreType.DMA,
            pltpu.SemaphoreType.DMA,
        ],
    )


@functools.lru_cache(maxsize=None)
def _deg_call(NB):
    NBC = NB // (NCORE * NSUB)
    RPW = NP // NSUB
    mesh = plsc.VectorSubcoreMesh(core_axis_name="c", subcore_axis_name="s")

    def body(dst_hbm, out_hbm, idx_d, ones_b, deg_sh):
        c = lax.axis_index("c")
        s = lax.axis_index("s")
        z = jnp.zeros((LANES,), F32)

        @pl.loop(0, BLK_D)
        def _(i):
            ones_b[i, pl.ds(0, LANES)] = z

        @pl.loop(0, RPW, step=BLK_D)
        def _(r):
            pltpu.sync_copy(ones_b, deg_sh.at[pl.ds(s * RPW + r, BLK_D)])

        o = jnp.ones((LANES,), F32)

        @pl.loop(0, BLK_D)
        def _(i):
            ones_b[i, pl.ds(0, LANES)] = o

        plsc.subcore_barrier()
        pltpu.sync_copy(dst_hbm.at[pl.ds((c * NSUB + s) * NBC, NBC)], idx_d)

        @pl.loop(0, NBC)
        def _(j):
            pltpu.sync_copy(ones_b, deg_sh.at[idx_d.at[j]], add=True)

        plsc.subcore_barrier()
        pltpu.sync_copy(deg_sh.at[pl.ds(s * RPW, RPW)],
                        out_hbm.at[c, pl.ds(s * RPW, RPW)])

    return pl.kernel(
        body,
        out_type=jax.ShapeDtypeStruct((NCORE, NP, LANES), F32),
        mesh=mesh,
        compiler_params=pltpu.CompilerParams(use_tc_tiling_on_sc=False),
        scratch_types=[
            pltpu.VMEM((NBC, BLK_D), jnp.int32),
            pltpu.VMEM((BLK_D, LANES), F32),
            pltpu.VMEM_SHARED((NP, LANES), F32),
        ],
    )


def _dinv_of(deg_block):
    return lax.rsqrt(deg_block[0, :, 0] + deg_block[1, :, 0] + 1.0)


def _matmul1(xp, W1p):
    def body(x_ref, w_ref, o_ref):
        o_ref[...] = jnp.dot(x_ref[...], w_ref[...], preferred_element_type=F32)

    return pl.pallas_call(
        body,
        grid=(NP // RB,),
        in_specs=[pl.BlockSpec((RB, D), lambda i: (i, 0)),
                  pl.BlockSpec((D, HP), lambda i: (0, 0))],
        out_specs=pl.BlockSpec((RB, HP), lambda i: (i, 0)),
        out_shape=jax.ShapeDtypeStruct((NP, HP), F32),
    )(xp, W1p)


def _scale_split(t1, degp, Wh):
    def body(t_ref, deg_ref, o_ref):
        dinv = _dinv_of(deg_ref[...])
        tv = t_ref[...] * dinv[:, None]
        o_ref[0] = tv[:, :Wh]
        o_ref[1] = tv[:, Wh:]

    return pl.pallas_call(
        body,
        grid=(NP // RB,),
        in_specs=[pl.BlockSpec((RB, 2 * Wh), lambda i: (i, 0)),
                  pl.BlockSpec((NCORE, RB, LANES), lambda i: (0, i, 0))],
        out_specs=pl.BlockSpec((NCORE, RB, Wh), lambda i: (0, i, 0)),
        out_shape=jax.ShapeDtypeStruct((NCORE, NP, Wh), F32),
    )(t1, degp)


def _fuse_layer(acc, tp, degp, b_row, Wn, Wh_out):
    Wh_in = tp.shape[2]

    def body(a_ref, t_ref, deg_ref, b_ref, w_ref, o_ref):
        dinv = _dinv_of(deg_ref[...])
        a = jnp.concatenate([a_ref[0], a_ref[1]], axis=1)
        t = jnp.concatenate([t_ref[0], t_ref[1]], axis=1)
        h = jnp.maximum((a + t) * dinv[:, None] + b_ref[...], 0.0)
        tn = jnp.dot(h, w_ref[...], preferred_element_type=F32) * dinv[:, None]
        o_ref[0] = tn[:, :Wh_out]
        o_ref[1] = tn[:, Wh_out:]

    return pl.pallas_call(
        body,
        grid=(NP // RB,),
        in_specs=[pl.BlockSpec((NCORE, RB, Wh_in), lambda i: (0, i, 0)),
                  pl.BlockSpec((NCORE, RB, Wh_in), lambda i: (0, i, 0)),
                  pl.BlockSpec((NCORE, RB, LANES), lambda i: (0, i, 0)),
                  pl.BlockSpec((1, 2 * Wh_in), lambda i: (0, 0)),
                  pl.BlockSpec((2 * Wh_in, 2 * Wh_out), lambda i: (0, 0))],
        out_specs=pl.BlockSpec((NCORE, RB, Wh_out), lambda i: (0, i, 0)),
        out_shape=jax.ShapeDtypeStruct((NCORE, NP, Wh_out), F32),
    )(acc, tp, degp, b_row, Wn)


def _final(acc, tp, degp, b_row, batch3, Wm, bm):
    nb = NP // RB
    Wh_in = tp.shape[2]
    OUTP = 2 * Wh_in

    def body(a_ref, t_ref, deg_ref, b_ref, bt_ref, wm_ref, bm_ref, o_ref,
             sums, counts):
        i = pl.program_id(0)

        @pl.when(i == 0)
        def _():
            sums[...] = jnp.zeros_like(sums)
            counts[...] = jnp.zeros_like(counts)

        dinv = _dinv_of(deg_ref[...])
        a = jnp.concatenate([a_ref[0], a_ref[1]], axis=1)
        t = jnp.concatenate([t_ref[0], t_ref[1]], axis=1)
        h = jnp.maximum((a + t) * dinv[:, None] + b_ref[...], 0.0)
        bt = bt_ref[0, 0, :]
        gid = lax.broadcasted_iota(jnp.int32, (G, RB), 0)
        ind = (bt[None, :] == gid).astype(F32)
        hh = h.astype(BF16)
        hl = (h - hh.astype(F32)).astype(BF16)
        indb = ind.astype(BF16)
        sums[...] += (jnp.dot(indb, hh, preferred_element_type=F32)
                      + jnp.dot(indb, hl, preferred_element_type=F32))
        counts[...] += jnp.sum(ind, axis=1)[None, :]

        @pl.when(i == nb - 1)
        def _():
            ro = sums[...] / jnp.maximum(counts[0, :], 1.0)[:, None]
            o_ref[...] = jnp.dot(ro, wm_ref[...],
                                 preferred_element_type=F32) + bm_ref[0, 0]

    return pl.pallas_call(
        body,
        grid=(nb,),
        in_specs=[pl.BlockSpec((NCORE, RB, Wh_in), lambda i: (0, i, 0)),
                  pl.BlockSpec((NCORE, RB, Wh_in), lambda i: (0, i, 0)),
                  pl.BlockSpec((NCORE, RB, LANES), lambda i: (0, i, 0)),
                  pl.BlockSpec((1, OUTP), lambda i: (0, 0)),
                  pl.BlockSpec((1, 1, RB), lambda i: (i, 0, 0)),
                  pl.BlockSpec((OUTP, 1), lambda i: (0, 0)),
                  pl.BlockSpec((1, 1), lambda i: (0, 0))],
        out_specs=pl.BlockSpec((G, 1), lambda i: (0, 0)),
        out_shape=jax.ShapeDtypeStruct((G, 1), F32),
        scratch_shapes=[pltpu.VMEM((G, OUTP), F32), pltpu.VMEM((1, G), F32)],
    )(acc, tp, degp, b_row, batch3, Wm, bm)


def kernel(x, edge_index, batch, W1, b1, W2, b2, W3, b3, Wm, bm):
    E = edge_index.shape[1]
    CH = NCORE * NSUB * 128 * 8
    EP = -(-E // CH) * CH
    NB64 = EP // 64
    NB128 = EP // 128

    src = edge_index[0].astype(jnp.int32)
    dst = edge_index[1].astype(jnp.int32)
    pad = jnp.full((EP - E,), N, jnp.int32)
    srcf = jnp.concatenate([src, pad])
    dstf = jnp.concatenate([dst, pad])
    srcp64 = srcf.reshape(NB64, 64)
    dstp64 = dstf.reshape(NB64, 64)
    srcp128 = srcf.reshape(NB128, 128)
    dstp128 = dstf.reshape(NB128, 128)
    src2_64 = jnp.stack([srcp64, srcp64 + NP])
    src2_128 = jnp.stack([srcp128, srcp128 + NP])

    xp = jnp.pad(x.astype(F32), ((0, NP - N), (0, 0)))
    W1p = jnp.pad(W1, ((0, 0), (0, HP - H)))
    W2p = jnp.pad(W2, ((0, HP - H), (0, HP - H)))
    W3p = jnp.pad(W3, ((0, HP - H), (0, 0)))
    b1r = jnp.pad(b1, (0, HP - H))[None, :]
    b2r = jnp.pad(b2, (0, HP - H))[None, :]
    b3r = b3[None, :]
    batchp = jnp.concatenate(
        [batch.astype(jnp.int32), jnp.full((NP - N,), G, jnp.int32)])
    batch3 = batchp.reshape(NP // RB, 1, RB)

    degp = _deg_call(NB128)(dstp128)
    t1 = _matmul1(xp, W1p)
    t1p = _scale_split(t1, degp, HP // 2)
    acc1 = _spmm_call(HP // 2, NB64, 64, 32)(
        t1p.reshape(NCORE * NP, HP // 2), src2_64, dstp64)
    t2p = _fuse_layer(acc1, t1p, degp, b1r, W2p, HP // 2)
    acc2 = _spmm_call(HP // 2, NB64, 64, 32)(
        t2p.reshape(NCORE * NP, HP // 2), src2_64, dstp64)
    t3p = _fuse_layer(acc2, t2p, degp, b2r, W3p, OUT // 2)
    acc3 = _spmm_call(OUT // 2, NB128, 128, 16)(
        t3p.reshape(NCORE * NP, OUT // 2), src2_128, dstp128)
    out = _final(acc3, t3p, degp, b3r, batch3, Wm, bm[None, :])
    return out[:, 0]
